# baseline (speedup 1.0000x reference)
import numpy as np
import jax
import jax.numpy as jnp
from jax.experimental import pallas as pl
from jax.experimental.pallas import tpu as pltpu

_N = 4096
_TB = 512
_NT = _N // _TB
_W3 = 0.33
_LOG2E = 1.4426950408889634
_C0 = -_LOG2E / (2.0 * 0.02 * 0.02)
_C1 = -_LOG2E / (2.0 * 0.08 * 0.08)
_C2 = -_LOG2E / (2.0 * 0.32 * 0.32)

_BLOCKS = ([(i, i) for i in range(_NT)]
           + [(i, j) for i in range(_NT) for j in range(i + 1, _NT)])
_NB = len(_BLOCKS)


def _net_body(ii, jj, uq, vs, fea,
              w00, b00, w01, b01, w02, b02,
              w10, b10, w11, b11, w12, b12,
              w20, b20, w21, b21, w22, b22,
              wr, br, out,
              fA, fB, fSB, acc, kst):
    def cc(src, w, b, dst_ref, relu=True):
        cout = w.shape[1]
        y = jnp.dot(src.astype(jnp.bfloat16), w[...],
                    preferred_element_type=jnp.float32) + b[...]
        if relu:
            y = jnp.maximum(y, 0.0)
        dst_ref[:, :cout] = y

    def load_f(cin, src_ref):
        fSB[:, :cin] = src_ref[:, :cin].astype(jnp.bfloat16)
        fSB[:, cin:cin + 1] = jnp.ones((_N, 1), jnp.bfloat16)
        acc[:, :cin + 1] = jnp.zeros((_N, cin + 1), jnp.float32)

    def normalize(cin, dst_ref):
        num = acc[:, :cin]
        rs = acc[:, cin:cin + 1]
        dst_ref[:, :cin] = (_W3 * num) / (_W3 * rs + 1e-8)

    def apply_blk(b, cin):
        ri = ii[b] * _TB
        rj = jj[b] * _TB
        kb = kst[pl.ds(b * _TB, _TB), :]
        acc[pl.ds(ri, _TB), :cin + 1] += jnp.dot(
            kb, fSB[pl.ds(rj, _TB), :cin + 1],
            preferred_element_type=jnp.float32)
        acc[pl.ds(rj, _TB), :cin + 1] += jax.lax.dot_general(
            kb, fSB[pl.ds(ri, _TB), :cin + 1],
            (((0,), (0,)), ((), ())),
            preferred_element_type=jnp.float32)

    def spatial(coef, cin, src_ref, dst_ref):
        load_f(cin, src_ref)

        def kblock(b):
            ri = ii[b] * _TB
            rj = jj[b] * _TB
            qd = jnp.dot(uq[pl.ds(ri, _TB), :], vs[:, pl.ds(rj, _TB)],
                         preferred_element_type=jnp.float32)
            t = jnp.minimum(qd * coef, 0.0)
            e = jnp.exp2(t)
            e2 = e * e
            e4 = e2 * e2
            e8 = e4 * e4
            e16 = e8 * e8
            k = (e + e4) + e16
            half = jnp.where(b < _NT, 0.5, 1.0).astype(jnp.float32)
            kst[pl.ds(b * _TB, _TB), :] = (k * half).astype(jnp.bfloat16)

        kblock(0)

        def step(b, c):
            apply_blk(b - 1, cin)
            kblock(b)
            return c

        jax.lax.fori_loop(1, _NB, step, 0)
        apply_blk(_NB - 1, cin)
        normalize(cin, dst_ref)

    def spatial_reuse(cin, src_ref, dst_ref):
        load_f(cin, src_ref)

        def step(b, c):
            apply_blk(b, cin)
            return c

        jax.lax.fori_loop(0, _NB, step, 0)
        normalize(cin, dst_ref)

    cc(fea[...], w00, b00, fA)
    cc(fA[:, :8], w01, b01, fB)
    spatial(_C0, 16, fB, fA)
    cc(fA[:, :16], w02, b02, fB)
    spatial(_C1, 16, fB, fA)
    cc(fA[:, :16], w10, b10, fB)
    cc(fB[:, :32], w11, b11, fA)
    spatial_reuse(32, fA, fB)
    cc(fB[:, :32], w12, b12, fA)
    spatial(_C2, 32, fA, fB)
    cc(fB[:, :32], w20, b20, fA)
    cc(fA[:, :64], w21, b21, fB)
    spatial_reuse(64, fB, fA)
    cc(fA[:, :64], w22, b22, fB)
    y = jnp.dot(fB[:, :64].astype(jnp.bfloat16), wr[...],
                preferred_element_type=jnp.float32) + br[...]
    out[...] = y


def kernel(pc1, feature1, W00, b00, W01, b01, W02, b02, W10, b10, W11, b11,
           W12, b12, W20, b20, W21, b21, W22, b22, Wr, br):
    pc = pc1[0]
    fea = feature1[0]
    nrm2 = jnp.sum(pc * pc, axis=1, keepdims=True)
    hi = nrm2.astype(jnp.bfloat16).astype(jnp.float32)
    lo = nrm2 - hi
    ones = jnp.ones((_N, 1), jnp.float32)
    zero = jnp.zeros((_N, 1), jnp.float32)
    uq = jnp.concatenate([pc, ones, ones, hi, lo, zero],
                         axis=1).astype(jnp.bfloat16)
    vs = jnp.concatenate([-2.0 * pc, hi, lo, ones, ones, zero],
                         axis=1).astype(jnp.bfloat16).T
    ii = jnp.asarray(np.array([p[0] for p in _BLOCKS], np.int32))
    jj = jnp.asarray(np.array([p[1] for p in _BLOCKS], np.int32))
    wts = []
    for w, b in ((W00, b00), (W01, b01), (W02, b02), (W10, b10), (W11, b11),
                 (W12, b12), (W20, b20), (W21, b21), (W22, b22), (Wr, br)):
        wts.append(w.T.astype(jnp.bfloat16))
        wts.append(b[None, :])
    out = pl.pallas_call(
        _net_body,
        out_shape=jax.ShapeDtypeStruct((_N, 32), jnp.float32),
        in_specs=[pl.BlockSpec(memory_space=pltpu.SMEM),
                  pl.BlockSpec(memory_space=pltpu.SMEM)] +
                 [pl.BlockSpec(memory_space=pltpu.VMEM)] * 23,
        out_specs=pl.BlockSpec(memory_space=pltpu.VMEM),
        scratch_shapes=[
            pltpu.VMEM((_N, 64), jnp.float32),
            pltpu.VMEM((_N, 64), jnp.float32),
            pltpu.VMEM((_N, 72), jnp.bfloat16),
            pltpu.VMEM((_N, 72), jnp.float32),
            pltpu.VMEM((_NB * _TB, _TB), jnp.bfloat16),
        ],
    )(ii, jj, uq, vs, fea, *wts)
    return out[None]


# speedup vs baseline: 1.0933x; 1.0933x over previous
import numpy as np
import jax
import jax.numpy as jnp
from jax.experimental import pallas as pl
from jax.experimental.pallas import tpu as pltpu

_N = 4096
_TB = 512
_NT = _N // _TB
_W3 = 0.33
_LOG2E = 1.4426950408889634
_C0 = -_LOG2E / (2.0 * 0.02 * 0.02)
_C1 = -_LOG2E / (2.0 * 0.08 * 0.08)
_C2 = -_LOG2E / (2.0 * 0.32 * 0.32)

_PAIRS = [(i, j) for i in range(_NT) for j in range(i + 1, _NT)]
_NP = len(_PAIRS)
_NB = _NT + _NP

_TRANS_RHS = (((1,), (1,)), ((), ()))
_TRANS_LHS = (((0,), (0,)), ((), ()))


def _net_body(ii, jj, pc1, fea1,
              w00, b00, w01, b01, w02, b02,
              w10, b10, w11, b11, w12, b12,
              w20, b20, w21, b21, w22, b22,
              wr, br, out,
              fA, fB, fSB, acc, uqs, vss, kst):
    pc = pc1[0]
    n2 = jnp.sum(pc * pc, axis=1, keepdims=True)
    hib = n2.astype(jnp.bfloat16)
    hi = hib.astype(jnp.float32)
    lob = (n2 - hi).astype(jnp.bfloat16)
    ob = jnp.ones((_N, 1), jnp.bfloat16)
    zb = jnp.zeros((_N, 1), jnp.bfloat16)
    uqs[:, 0:3] = pc.astype(jnp.bfloat16)
    uqs[:, 3:4] = ob
    uqs[:, 4:5] = ob
    uqs[:, 5:6] = hib
    uqs[:, 6:7] = lob
    uqs[:, 7:8] = zb
    vss[:, 0:3] = (-2.0 * pc).astype(jnp.bfloat16)
    vss[:, 3:4] = hib
    vss[:, 4:5] = lob
    vss[:, 5:6] = ob
    vss[:, 6:7] = ob
    vss[:, 7:8] = zb

    def cc(src, w, b, dst_ref, relu=True):
        cout = w.shape[0]
        y = jax.lax.dot_general(
            src.astype(jnp.bfloat16), w[...].astype(jnp.bfloat16),
            _TRANS_RHS, preferred_element_type=jnp.float32)
        y = y + jax.lax.broadcast_in_dim(b[...], (1, cout), (1,))
        if relu:
            y = jnp.maximum(y, 0.0)
        dst_ref[:, :cout] = y

    def load_f(cin, src_ref):
        fSB[:, :cin] = src_ref[:, :cin].astype(jnp.bfloat16)
        fSB[:, cin:cin + 1] = jnp.ones((_N, 1), jnp.bfloat16)
        acc[:, :cin + 1] = jnp.zeros((_N, cin + 1), jnp.float32)

    def normalize(cin, dst_ref):
        num = acc[:, :cin]
        rs = acc[:, cin:cin + 1]
        dst_ref[:, :cin] = (_W3 * num) / (_W3 * rs + 1e-8)

    def apply_blk(kb, ri, rj, cin, both):
        acc[pl.ds(ri, _TB), :cin + 1] += jnp.dot(
            kb, fSB[pl.ds(rj, _TB), :cin + 1],
            preferred_element_type=jnp.float32)
        if both:
            acc[pl.ds(rj, _TB), :cin + 1] += jax.lax.dot_general(
                kb, fSB[pl.ds(ri, _TB), :cin + 1],
                _TRANS_LHS, preferred_element_type=jnp.float32)

    def spatial(coef, cin, src_ref, dst_ref, store):
        load_f(cin, src_ref)

        def kblock(ri, rj):
            qd = jax.lax.dot_general(
                uqs[pl.ds(ri, _TB), :], vss[pl.ds(rj, _TB), :],
                _TRANS_RHS, preferred_element_type=jnp.float32)
            t = jnp.minimum(qd * coef, 0.0)
            e = jnp.exp2(t)
            e2 = e * e
            e4 = e2 * e2
            e8 = e4 * e4
            e16 = e8 * e8
            k = (e + e4) + e16
            return k.astype(jnp.bfloat16)

        def diag(i, c):
            r0 = i * _TB
            kb = kblock(r0, r0)
            if store:
                kst[pl.ds(i * _TB, _TB), :] = kb
            apply_blk(kb, r0, r0, cin, False)
            return c

        def offd(p, c):
            ri = ii[p] * _TB
            rj = jj[p] * _TB
            kb = kblock(ri, rj)
            if store:
                kst[pl.ds((_NT + p) * _TB, _TB), :] = kb
            apply_blk(kb, ri, rj, cin, True)
            return c

        jax.lax.fori_loop(0, _NT, diag, 0)
        jax.lax.fori_loop(0, _NP, offd, 0)
        normalize(cin, dst_ref)

    def spatial_reuse(cin, src_ref, dst_ref):
        load_f(cin, src_ref)

        def diag(i, c):
            r0 = i * _TB
            kb = kst[pl.ds(i * _TB, _TB), :]
            apply_blk(kb, r0, r0, cin, False)
            return c

        def offd(p, c):
            ri = ii[p] * _TB
            rj = jj[p] * _TB
            kb = kst[pl.ds((_NT + p) * _TB, _TB), :]
            apply_blk(kb, ri, rj, cin, True)
            return c

        jax.lax.fori_loop(0, _NT, diag, 0)
        jax.lax.fori_loop(0, _NP, offd, 0)
        normalize(cin, dst_ref)

    cc(fea1[0], w00, b00, fA)
    cc(fA[:, :8], w01, b01, fB)
    spatial(_C0, 16, fB, fA, False)
    cc(fA[:, :16], w02, b02, fB)
    spatial(_C1, 16, fB, fA, True)
    cc(fA[:, :16], w10, b10, fB)
    cc(fB[:, :32], w11, b11, fA)
    spatial_reuse(32, fA, fB)
    cc(fB[:, :32], w12, b12, fA)
    spatial(_C2, 32, fA, fB, True)
    cc(fB[:, :32], w20, b20, fA)
    cc(fA[:, :64], w21, b21, fB)
    spatial_reuse(64, fB, fA)
    cc(fA[:, :64], w22, b22, fB)
    y = jax.lax.dot_general(
        fB[:, :64].astype(jnp.bfloat16), wr[...].astype(jnp.bfloat16),
        _TRANS_RHS, preferred_element_type=jnp.float32)
    out[0] = y + jax.lax.broadcast_in_dim(br[...], (1, 32), (1,))


def kernel(pc1, feature1, W00, b00, W01, b01, W02, b02, W10, b10, W11, b11,
           W12, b12, W20, b20, W21, b21, W22, b22, Wr, br):
    ii = jnp.asarray(np.array([p[0] for p in _PAIRS], np.int32))
    jj = jnp.asarray(np.array([p[1] for p in _PAIRS], np.int32))
    return pl.pallas_call(
        _net_body,
        out_shape=jax.ShapeDtypeStruct((1, _N, 32), jnp.float32),
        in_specs=[pl.BlockSpec(memory_space=pltpu.SMEM),
                  pl.BlockSpec(memory_space=pltpu.SMEM)] +
                 [pl.BlockSpec(memory_space=pltpu.VMEM)] * 22,
        out_specs=pl.BlockSpec(memory_space=pltpu.VMEM),
        scratch_shapes=[
            pltpu.VMEM((_N, 64), jnp.float32),
            pltpu.VMEM((_N, 64), jnp.float32),
            pltpu.VMEM((_N, 72), jnp.bfloat16),
            pltpu.VMEM((_N, 72), jnp.float32),
            pltpu.VMEM((_N, 8), jnp.bfloat16),
            pltpu.VMEM((_N, 8), jnp.bfloat16),
            pltpu.VMEM((_NB * _TB, _TB), jnp.bfloat16),
        ],
    )(ii, jj, pc1, feature1, W00, b00, W01, b01, W02, b02, W10, b10,
      W11, b11, W12, b12, W20, b20, W21, b21, W22, b22, Wr, br)


# speedup vs baseline: 1.3969x; 1.2777x over previous
import numpy as np
import jax
import jax.numpy as jnp
from jax.experimental import pallas as pl
from jax.experimental.pallas import tpu as pltpu

_N = 4096
_TB = 1024
_NT = _N // _TB
_W3 = 0.33
_LOG2E = 1.4426950408889634
_C0 = -_LOG2E / (2.0 * 0.02 * 0.02)
_C1 = -_LOG2E / (2.0 * 0.08 * 0.08)
_C2 = -_LOG2E / (2.0 * 0.32 * 0.32)

_PAIRS = [(i, j) for i in range(_NT) for j in range(i + 1, _NT)]
_NP = len(_PAIRS)
_NB = _NT + _NP

_TRANS_RHS = (((1,), (1,)), ((), ()))
_TRANS_LHS = (((0,), (0,)), ((), ()))


def _net_body(ii, jj, pc1, fea1,
              w00, b00, w01, b01, w02, b02,
              w10, b10, w11, b11, w12, b12,
              w20, b20, w21, b21, w22, b22,
              wr, br, out,
              fA, fB, fSB, acc, uqs, vss, kst):
    pc = pc1[0]
    n2 = jnp.sum(pc * pc, axis=1, keepdims=True)
    hib = n2.astype(jnp.bfloat16)
    hi = hib.astype(jnp.float32)
    lob = (n2 - hi).astype(jnp.bfloat16)
    ob = jnp.ones((_N, 1), jnp.bfloat16)
    zb = jnp.zeros((_N, 1), jnp.bfloat16)
    uqs[:, 0:3] = pc.astype(jnp.bfloat16)
    uqs[:, 3:4] = ob
    uqs[:, 4:5] = ob
    uqs[:, 5:6] = hib
    uqs[:, 6:7] = lob
    uqs[:, 7:8] = zb
    vss[:, 0:3] = (-2.0 * pc).astype(jnp.bfloat16)
    vss[:, 3:4] = hib
    vss[:, 4:5] = lob
    vss[:, 5:6] = ob
    vss[:, 6:7] = ob
    vss[:, 7:8] = zb

    def cc(src, w, b, dst_ref, relu=True):
        cout = w.shape[0]
        y = jax.lax.dot_general(
            src.astype(jnp.bfloat16), w[...].astype(jnp.bfloat16),
            _TRANS_RHS, preferred_element_type=jnp.float32)
        y = y + jax.lax.broadcast_in_dim(b[...], (1, cout), (1,))
        if relu:
            y = jnp.maximum(y, 0.0)
        dst_ref[:, :cout] = y

    def load_f(cin, src_ref):
        fSB[:, :cin] = src_ref[:, :cin].astype(jnp.bfloat16)
        fSB[:, cin:cin + 1] = jnp.ones((_N, 1), jnp.bfloat16)
        acc[:, :cin + 1] = jnp.zeros((_N, cin + 1), jnp.float32)

    def normalize(cin, dst_ref):
        num = acc[:, :cin]
        rs = acc[:, cin:cin + 1]
        dst_ref[:, :cin] = (_W3 * num) / (_W3 * rs + 1e-8)

    def apply_blk(kb, ri, rj, cin, both):
        acc[pl.ds(ri, _TB), :cin + 1] += jnp.dot(
            kb, fSB[pl.ds(rj, _TB), :cin + 1],
            preferred_element_type=jnp.float32)
        if both:
            acc[pl.ds(rj, _TB), :cin + 1] += jax.lax.dot_general(
                kb, fSB[pl.ds(ri, _TB), :cin + 1],
                _TRANS_LHS, preferred_element_type=jnp.float32)

    def spatial(coef, cin, src_ref, dst_ref, store):
        load_f(cin, src_ref)

        def kblock(ri, rj):
            qd = jax.lax.dot_general(
                uqs[pl.ds(ri, _TB), :], vss[pl.ds(rj, _TB), :],
                _TRANS_RHS, preferred_element_type=jnp.float32)
            t = jnp.minimum(qd * coef, 0.0)
            e = jnp.exp2(t)
            e2 = e * e
            e4 = e2 * e2
            e8 = e4 * e4
            e16 = e8 * e8
            k = (e + e4) + e16
            return k.astype(jnp.bfloat16)

        def diag(i, c):
            r0 = i * _TB
            kb = kblock(r0, r0)
            if store:
                kst[pl.ds(i * _TB, _TB), :] = kb
            apply_blk(kb, r0, r0, cin, False)
            return c

        def offd(p, c):
            ri = ii[p] * _TB
            rj = jj[p] * _TB
            kb = kblock(ri, rj)
            if store:
                kst[pl.ds((_NT + p) * _TB, _TB), :] = kb
            apply_blk(kb, ri, rj, cin, True)
            return c

        jax.lax.fori_loop(0, _NT, diag, 0)
        jax.lax.fori_loop(0, _NP, offd, 0)
        normalize(cin, dst_ref)

    def spatial_reuse(cin, src_ref, dst_ref):
        load_f(cin, src_ref)

        def diag(i, c):
            r0 = i * _TB
            kb = kst[pl.ds(i * _TB, _TB), :]
            apply_blk(kb, r0, r0, cin, False)
            return c

        def offd(p, c):
            ri = ii[p] * _TB
            rj = jj[p] * _TB
            kb = kst[pl.ds((_NT + p) * _TB, _TB), :]
            apply_blk(kb, ri, rj, cin, True)
            return c

        jax.lax.fori_loop(0, _NT, diag, 0)
        jax.lax.fori_loop(0, _NP, offd, 0)
        normalize(cin, dst_ref)

    cc(fea1[0], w00, b00, fA)
    cc(fA[:, :8], w01, b01, fB)
    spatial(_C0, 16, fB, fA, False)
    cc(fA[:, :16], w02, b02, fB)
    spatial(_C1, 16, fB, fA, True)
    cc(fA[:, :16], w10, b10, fB)
    cc(fB[:, :32], w11, b11, fA)
    spatial_reuse(32, fA, fB)
    cc(fB[:, :32], w12, b12, fA)
    spatial(_C2, 32, fA, fB, True)
    cc(fB[:, :32], w20, b20, fA)
    cc(fA[:, :64], w21, b21, fB)
    spatial_reuse(64, fB, fA)
    cc(fA[:, :64], w22, b22, fB)
    y = jax.lax.dot_general(
        fB[:, :64].astype(jnp.bfloat16), wr[...].astype(jnp.bfloat16),
        _TRANS_RHS, preferred_element_type=jnp.float32)
    out[0] = y + jax.lax.broadcast_in_dim(br[...], (1, 32), (1,))


def kernel(pc1, feature1, W00, b00, W01, b01, W02, b02, W10, b10, W11, b11,
           W12, b12, W20, b20, W21, b21, W22, b22, Wr, br):
    ii = jnp.asarray(np.array([p[0] for p in _PAIRS], np.int32))
    jj = jnp.asarray(np.array([p[1] for p in _PAIRS], np.int32))
    return pl.pallas_call(
        _net_body,
        out_shape=jax.ShapeDtypeStruct((1, _N, 32), jnp.float32),
        in_specs=[pl.BlockSpec(memory_space=pltpu.SMEM),
                  pl.BlockSpec(memory_space=pltpu.SMEM)] +
                 [pl.BlockSpec(memory_space=pltpu.VMEM)] * 22,
        out_specs=pl.BlockSpec(memory_space=pltpu.VMEM),
        scratch_shapes=[
            pltpu.VMEM((_N, 64), jnp.float32),
            pltpu.VMEM((_N, 64), jnp.float32),
            pltpu.VMEM((_N, 72), jnp.bfloat16),
            pltpu.VMEM((_N, 72), jnp.float32),
            pltpu.VMEM((_N, 8), jnp.bfloat16),
            pltpu.VMEM((_N, 8), jnp.bfloat16),
            pltpu.VMEM((_NB * _TB, _TB), jnp.bfloat16),
        ],
    )(ii, jj, pc1, feature1, W00, b00, W01, b01, W02, b02, W10, b10,
      W11, b11, W12, b12, W20, b20, W21, b21, W22, b22, Wr, br)
